# baseline (speedup 1.0000x reference)
import functools

import jax
import jax.numpy as jnp
from jax import lax
from jax.experimental import pallas as pl
from jax.experimental.pallas import tpu as pltpu
from jax.experimental.pallas import tpu_sc as plsc

_NC = 2
_NS = 16
_NW = _NC * _NS
_L = 16


def _make_sc_pairdma(B, D):
    b_per_w = B // _NW
    pairs_per_w = b_per_w // 2
    ngroups = b_per_w // _L
    mesh = plsc.VectorSubcoreMesh(core_axis_name="c", subcore_axis_name="s")

    @functools.partial(
        pl.kernel,
        mesh=mesh,
        out_type=jax.ShapeDtypeStruct((B // 2, 2 * D), jnp.float32),
        scratch_types=[
            pltpu.VMEM((4, 2 * D), jnp.float32),
            pltpu.VMEM((b_per_w,), jnp.int32),
            pltpu.SemaphoreType.DMA,
        ],
    )
    def k(table_hbm, idx_hbm, out_hbm, ptab, ids_v, sem):
        wid = lax.axis_index("s") * _NC + lax.axis_index("c")
        base = wid * b_per_w
        pbase = wid * pairs_per_w
        for c in range(4):
            pltpu.sync_copy(table_hbm.at[c // 2], ptab.at[c, pl.ds(0, D)])
            pltpu.sync_copy(table_hbm.at[c % 2], ptab.at[c, pl.ds(D, D)])
        pltpu.sync_copy(idx_hbm.at[pl.ds(base, b_per_w)], ids_v)

        def fire_group(g):
            p0 = g * _L
            idsv = ids_v[pl.ds(p0, _L)]
            for j in range(_L // 2):
                c = idsv[2 * j] * 2 + idsv[2 * j + 1]
                pltpu.async_copy(
                    ptab.at[pl.ds(c, 1)],
                    out_hbm.at[pl.ds(pbase + g * (_L // 2) + j, 1)],
                    sem,
                )

        def drain_group():
            for _ in range(_L // 2):
                pltpu.make_async_copy(
                    ptab.at[pl.ds(0, 1)],
                    out_hbm.at[pl.ds(pbase, 1)],
                    sem,
                ).wait()

        def body(i, carry):
            fire_group(i)
            drain_group()
            return carry

        lax.fori_loop(0, ngroups, body, 0)

    return k


def kernel(speaker_id, emb_table):
    b, t = speaker_id.shape
    _, d = emb_table.shape
    flat_ids = speaker_id.reshape(b * t)
    fn = _make_sc_pairdma(b * t, d)
    out = fn(emb_table, flat_ids)
    return out.reshape(b, t, d)


# speedup vs baseline: 3.6814x; 3.6814x over previous
import functools

import jax
import jax.numpy as jnp
from jax import lax
from jax.experimental import pallas as pl
from jax.experimental.pallas import tpu as pltpu
from jax.experimental.pallas import tpu_sc as plsc

_NC = 2
_NS = 16
_NW = _NC * _NS
_L = 16


def _make_sc_rowdma(B, D):
    b_per_w = B // _NW
    mesh = plsc.VectorSubcoreMesh(core_axis_name="c", subcore_axis_name="s")

    @functools.partial(
        pl.kernel,
        mesh=mesh,
        out_type=jax.ShapeDtypeStruct((B, D), jnp.float32),
        scratch_types=[
            pltpu.VMEM((2, D), jnp.float32),
            pltpu.VMEM((b_per_w,), jnp.int32),
            pltpu.SemaphoreType.DMA,
        ],
    )
    def k(table_hbm, idx_hbm, out_hbm, table_v, ids_v, sem):
        wid = lax.axis_index("s") * _NC + lax.axis_index("c")
        base = wid * b_per_w
        pltpu.sync_copy(table_hbm, table_v)
        pltpu.sync_copy(idx_hbm.at[pl.ds(base, b_per_w)], ids_v)

        def body(g, carry):
            p0 = g * _L
            idsv = ids_v[pl.ds(p0, _L)]
            for j in range(_L):
                row = idsv[j]
                pltpu.async_copy(
                    table_v.at[pl.ds(row, 1)],
                    out_hbm.at[pl.ds(base + p0 + j, 1)],
                    sem,
                )
            for j in range(_L):
                pltpu.make_async_copy(
                    table_v.at[pl.ds(0, 1)],
                    out_hbm.at[pl.ds(base + p0 + j, 1)],
                    sem,
                ).wait()
            return carry

        lax.fori_loop(0, b_per_w // _L, body, 0)

    return k


def kernel(speaker_id, emb_table):
    b, t = speaker_id.shape
    _, d = emb_table.shape
    flat_ids = speaker_id.reshape(b * t)
    fn = _make_sc_rowdma(b * t, d)
    out = fn(emb_table, flat_ids)
    return out.reshape(b, t, d)
